# baseline (device time: 190271 ns/iter reference)
import os

import jax
import jax.numpy as jnp
from jax import lax
from jax.experimental import pallas as pl
from jax.experimental.pallas import tpu as pltpu

N_DEV = 4
BB = 2
_ABL = os.environ.get("KERNEL_ABL", "full")


def kernel(Q, K, V):
    b, q, h, d = Q.shape
    kv = K.shape[1]
    hd = h * d
    n_steps = b // BB
    scale = d ** -0.5

    mask = (
        jnp.arange(hd, dtype=jnp.int32) // d
        == jnp.arange(h, dtype=jnp.int32)[:, None]
    ).astype(jnp.float32)
    qbd = mask.T[None, :, :] * (Q[:, 0].reshape(b, hd) * scale)[:, :, None]
    k2 = K.reshape(b, kv, hd)
    v2 = V.reshape(b, kv, hd)

    def body(qbd_ref, k_ref, v_ref, m_ref, out_ref,
             o_big, o_comm, l_comm, send_a, recv_a, send_b, recv_b):
        step = pl.program_id(0)

        if _ABL == "dma":
            l_comm[0, pl.ds(step * BB, BB)] = k_ref[:, 0, :h] + v_ref[:, 0, :h]
        else:
            s = lax.dot_general(
                k_ref[...], qbd_ref[...],
                (((2,), (1,)), ((0,), (0,))),
                preferred_element_type=jnp.float32,
            )
            p = jnp.exp(s)
            l_comm[0, pl.ds(step * BB, BB)] = jnp.sum(p, axis=1)
            o_big[pl.ds(step * BB, BB)] = lax.dot_general(
                p, v_ref[...],
                (((1,), (1,)), ((0,), (0,))),
                preferred_element_type=jnp.float32,
            )

        @pl.when(step == n_steps - 1)
        def _():
            o_comm[0] = jnp.sum(o_big[...] * m_ref[...][None, :, :], axis=1)
            my = lax.axis_index("i")
            pa = my ^ 1
            pb = 3 - my

            barrier = pltpu.get_barrier_semaphore()
            for nbr in (pa, pb):
                pl.semaphore_signal(
                    barrier, inc=1,
                    device_id=(nbr,), device_id_type=pl.DeviceIdType.MESH,
                )
            pl.semaphore_wait(barrier, 2)

            o_rdma_a = pltpu.make_async_remote_copy(
                src_ref=o_comm.at[0], dst_ref=o_comm.at[1],
                send_sem=send_a.at[0], recv_sem=recv_a.at[0],
                device_id=(pa,), device_id_type=pl.DeviceIdType.MESH,
            )
            l_rdma_a = pltpu.make_async_remote_copy(
                src_ref=l_comm.at[0], dst_ref=l_comm.at[1],
                send_sem=send_a.at[1], recv_sem=recv_a.at[1],
                device_id=(pa,), device_id_type=pl.DeviceIdType.MESH,
            )
            o_rdma_a.start()
            l_rdma_a.start()
            o_rdma_a.wait()
            l_rdma_a.wait()

            o_comm[2] = o_comm[0] + o_comm[1]
            l_comm[2] = l_comm[0] + l_comm[1]

            o_rdma_b = pltpu.make_async_remote_copy(
                src_ref=o_comm.at[2], dst_ref=o_comm.at[3],
                send_sem=send_b.at[0], recv_sem=recv_b.at[0],
                device_id=(pb,), device_id_type=pl.DeviceIdType.MESH,
            )
            l_rdma_b = pltpu.make_async_remote_copy(
                src_ref=l_comm.at[2], dst_ref=l_comm.at[3],
                send_sem=send_b.at[1], recv_sem=recv_b.at[1],
                device_id=(pb,), device_id_type=pl.DeviceIdType.MESH,
            )
            o_rdma_b.start()
            l_rdma_b.start()
            o_rdma_b.wait()
            l_rdma_b.wait()

            o_tot = o_comm[2] + o_comm[3]
            l_tot = l_comm[2] + l_comm[3]
            l_wide = lax.dot_general(
                l_tot, m_ref[...],
                (((1,), (0,)), ((), ())),
                preferred_element_type=jnp.float32,
            )
            out_ref[...] = o_tot / l_wide

    out = pl.pallas_call(
        body,
        grid=(n_steps,),
        in_specs=[
            pl.BlockSpec((BB, hd, h), lambda i: (i, 0, 0)),
            pl.BlockSpec((BB, kv, hd), lambda i: (i, 0, 0)),
            pl.BlockSpec((BB, kv, hd), lambda i: (i, 0, 0)),
            pl.BlockSpec((h, hd), lambda i: (0, 0)),
        ],
        out_specs=pl.BlockSpec((b, hd), lambda i: (0, 0)),
        out_shape=jax.ShapeDtypeStruct((b, hd), jnp.float32),
        scratch_shapes=[
            pltpu.VMEM((b, h, hd), jnp.float32),
            pltpu.VMEM((4, b, hd), jnp.float32),
            pltpu.VMEM((4, b, h), jnp.float32),
            pltpu.SemaphoreType.DMA((2,)),
            pltpu.SemaphoreType.DMA((2,)),
            pltpu.SemaphoreType.DMA((2,)),
            pltpu.SemaphoreType.DMA((2,)),
        ],
        compiler_params=pltpu.CompilerParams(
            dimension_semantics=("arbitrary",),
            collective_id=0,
            vmem_limit_bytes=48 * 1024 * 1024,
        ),
    )(qbd, k2, v2, mask)
    return out.reshape(b, q, h, d)


# device time: 189891 ns/iter; 1.0020x vs baseline; 1.0020x over previous
import jax
import jax.numpy as jnp
from jax import lax
from jax.experimental import pallas as pl
from jax.experimental.pallas import tpu as pltpu

N_DEV = 4
BB = 2


def kernel(Q, K, V):
    b, q, h, d = Q.shape
    kv = K.shape[1]
    hd = h * d
    n_chunks = b // BB
    scale = d ** -0.5

    mask = (
        jnp.arange(hd, dtype=jnp.int32) // d
        == jnp.arange(h, dtype=jnp.int32)[:, None]
    ).astype(jnp.float32)
    qbd = mask.T[None, :, :] * (Q[:, 0].reshape(b, hd) * scale)[:, :, None]
    k2 = K.reshape(b, kv, hd)
    v2 = V.reshape(b, kv, hd)

    def body(qbd_ref, k_hbm, v_hbm, m_ref, out_ref,
             kbuf, vbuf, ksems, vsems,
             o_big, o_comm, l_comm, send_a, recv_a, send_b, recv_b):

        def chunk_copies(c, slot):
            cps = []
            for r in range(BB):
                cps.append(pltpu.make_async_copy(
                    k_hbm.at[c * BB + r], kbuf.at[slot, r], ksems.at[slot, r]))
                cps.append(pltpu.make_async_copy(
                    v_hbm.at[c * BB + r], vbuf.at[slot, r], vsems.at[slot, r]))
            return cps

        for cp in chunk_copies(0, 0):
            cp.start()

        for c in range(n_chunks):
            slot = c % 2
            if c + 1 < n_chunks:
                for cp in chunk_copies(c + 1, (c + 1) % 2):
                    cp.start()
            for cp in chunk_copies(c, slot):
                cp.wait()

            s = lax.dot_general(
                kbuf[slot], qbd_ref[pl.ds(c * BB, BB)],
                (((2,), (1,)), ((0,), (0,))),
                preferred_element_type=jnp.float32,
            )
            p = jnp.exp(s)
            l_comm[0, pl.ds(c * BB, BB)] = jnp.sum(p, axis=1)
            o_big[pl.ds(c * BB, BB)] = lax.dot_general(
                p, vbuf[slot],
                (((1,), (1,)), ((0,), (0,))),
                preferred_element_type=jnp.float32,
            )

        o_comm[0] = jnp.sum(o_big[...] * m_ref[...][None, :, :], axis=1)

        my = lax.axis_index("i")
        pa = my ^ 1
        pb = 3 - my

        barrier = pltpu.get_barrier_semaphore()
        for nbr in (pa, pb):
            pl.semaphore_signal(
                barrier, inc=1,
                device_id=(nbr,), device_id_type=pl.DeviceIdType.MESH,
            )
        pl.semaphore_wait(barrier, 2)

        o_rdma_a = pltpu.make_async_remote_copy(
            src_ref=o_comm.at[0], dst_ref=o_comm.at[1],
            send_sem=send_a.at[0], recv_sem=recv_a.at[0],
            device_id=(pa,), device_id_type=pl.DeviceIdType.MESH,
        )
        l_rdma_a = pltpu.make_async_remote_copy(
            src_ref=l_comm.at[0], dst_ref=l_comm.at[1],
            send_sem=send_a.at[1], recv_sem=recv_a.at[1],
            device_id=(pa,), device_id_type=pl.DeviceIdType.MESH,
        )
        o_rdma_a.start()
        l_rdma_a.start()
        o_rdma_a.wait()
        l_rdma_a.wait()

        o_comm[2] = o_comm[0] + o_comm[1]
        l_comm[2] = l_comm[0] + l_comm[1]

        o_rdma_b = pltpu.make_async_remote_copy(
            src_ref=o_comm.at[2], dst_ref=o_comm.at[3],
            send_sem=send_b.at[0], recv_sem=recv_b.at[0],
            device_id=(pb,), device_id_type=pl.DeviceIdType.MESH,
        )
        l_rdma_b = pltpu.make_async_remote_copy(
            src_ref=l_comm.at[2], dst_ref=l_comm.at[3],
            send_sem=send_b.at[1], recv_sem=recv_b.at[1],
            device_id=(pb,), device_id_type=pl.DeviceIdType.MESH,
        )
        o_rdma_b.start()
        l_rdma_b.start()
        o_rdma_b.wait()
        l_rdma_b.wait()

        o_tot = o_comm[2] + o_comm[3]
        l_tot = l_comm[2] + l_comm[3]
        l_wide = lax.dot_general(
            l_tot, m_ref[...],
            (((1,), (0,)), ((), ())),
            preferred_element_type=jnp.float32,
        )
        out_ref[...] = o_tot / l_wide

    out = pl.pallas_call(
        body,
        in_specs=[
            pl.BlockSpec(memory_space=pltpu.MemorySpace.VMEM),
            pl.BlockSpec(memory_space=pl.ANY),
            pl.BlockSpec(memory_space=pl.ANY),
            pl.BlockSpec(memory_space=pltpu.MemorySpace.VMEM),
        ],
        out_specs=pl.BlockSpec(memory_space=pltpu.MemorySpace.VMEM),
        out_shape=jax.ShapeDtypeStruct((b, hd), jnp.float32),
        scratch_shapes=[
            pltpu.VMEM((2, BB, kv, hd), jnp.float32),
            pltpu.VMEM((2, BB, kv, hd), jnp.float32),
            pltpu.SemaphoreType.DMA((2, BB)),
            pltpu.SemaphoreType.DMA((2, BB)),
            pltpu.VMEM((b, h, hd), jnp.float32),
            pltpu.VMEM((4, b, hd), jnp.float32),
            pltpu.VMEM((4, b, h), jnp.float32),
            pltpu.SemaphoreType.DMA((2,)),
            pltpu.SemaphoreType.DMA((2,)),
            pltpu.SemaphoreType.DMA((2,)),
            pltpu.SemaphoreType.DMA((2,)),
        ],
        compiler_params=pltpu.CompilerParams(
            collective_id=0,
            vmem_limit_bytes=48 * 1024 * 1024,
        ),
    )(qbd, k2, v2, mask)
    return out.reshape(b, q, h, d)
